# baseline (speedup 1.0000x reference)
import functools

import jax
import jax.numpy as jnp
from jax import lax
from jax.experimental import pallas as pl
from jax.experimental.pallas import tpu as pltpu
from jax.experimental.pallas import tpu_sc as plsc

_LN_EPS = 1e-3

_NC = 2
_NS = 16
_NW = _NC * _NS


def _sc_embed_body(n_per_w, chunk, seq_len,
                   word_hbm, tt_hbm, pos_hbm, idx_hbm, tti_hbm, out_hbm,
                   idx_v, tti_v, buf0, buf1,
                   ps0, ps1, gs0, gs1, ts0, ts1, ss0, ss1):
    wid = lax.axis_index("s") * _NC + lax.axis_index("c")
    base = wid * n_per_w
    pos_start = lax.rem(base, seq_len)
    pltpu.sync_copy(idx_hbm.at[pl.ds(base, n_per_w)], idx_v)
    pltpu.sync_copy(tti_hbm.at[pl.ds(base, n_per_w)], tti_v)

    bufs = (buf0, buf1)
    psems = (ps0, ps1)
    gsems = (gs0, gs1)
    tsems = (ts0, ts1)
    ssems = (ss0, ss1)
    nchunks = n_per_w // chunk
    pf = {}
    wa = {}
    ta = {}
    sc = {}

    def posfill(c):
        pf[c] = pltpu.async_copy(
            pos_hbm.at[pl.ds(pos_start + c * chunk, chunk)],
            bufs[c % 2], psems[c % 2])

    def adds(c):
        wa[c] = pltpu.async_copy(
            word_hbm.at[idx_v.at[pl.ds(c * chunk, chunk)]],
            bufs[c % 2], gsems[c % 2], add=True)
        ta[c] = pltpu.async_copy(
            tt_hbm.at[tti_v.at[pl.ds(c * chunk, chunk)]],
            bufs[c % 2], tsems[c % 2], add=True)

    posfill(0)
    pf[0].wait()
    adds(0)
    for c in range(nchunks):
        wa[c].wait()
        ta[c].wait()
        sc[c] = pltpu.async_copy(
            bufs[c % 2], out_hbm.at[pl.ds(base + c * chunk, chunk)],
            ssems[c % 2])
        if c + 1 < nchunks:
            if c - 1 >= 0:
                sc[c - 1].wait()
            posfill(c + 1)
            pf[c + 1].wait()
            adds(c + 1)
    for c in range(max(0, nchunks - 2), nchunks):
        sc[c].wait()


def _sc_embed(word_table, tt_table, pos_table, ids_flat, tti_flat):
    n = ids_flat.shape[0]
    e = word_table.shape[1]
    seq_len = pos_table.shape[0]
    n_per_w = n // _NW
    chunk = min(256, n_per_w)
    mesh = plsc.VectorSubcoreMesh(core_axis_name="c", subcore_axis_name="s")
    return pl.kernel(
        functools.partial(_sc_embed_body, n_per_w, chunk, seq_len),
        out_type=jax.ShapeDtypeStruct((n, e), jnp.float32),
        mesh=mesh,
        scratch_types=[
            pltpu.VMEM((n_per_w,), jnp.int32),
            pltpu.VMEM((n_per_w,), jnp.int32),
            pltpu.VMEM((chunk, e), jnp.float32),
            pltpu.VMEM((chunk, e), jnp.float32),
            pltpu.SemaphoreType.DMA,
            pltpu.SemaphoreType.DMA,
            pltpu.SemaphoreType.DMA,
            pltpu.SemaphoreType.DMA,
            pltpu.SemaphoreType.DMA,
            pltpu.SemaphoreType.DMA,
            pltpu.SemaphoreType.DMA,
            pltpu.SemaphoreType.DMA,
        ],
    )(word_table, tt_table, pos_table, ids_flat, tti_flat)


def _tc_ln_body(rows_ref, g_ref, b_ref, o_ref):
    x = rows_ref[...]
    mean = jnp.mean(x, axis=-1, keepdims=True)
    xc = x - mean
    var = jnp.mean(xc * xc, axis=-1, keepdims=True)
    norm = xc * lax.rsqrt(var + _LN_EPS)
    o_ref[...] = norm * g_ref[...] + b_ref[...]


def _tc_ln(rows, gamma, beta):
    n, e = rows.shape
    t = 2048
    grid = (n // t,)
    return pl.pallas_call(
        _tc_ln_body,
        grid=grid,
        in_specs=[
            pl.BlockSpec((t, e), lambda g: (g, 0)),
            pl.BlockSpec((1, e), lambda g: (0, 0)),
            pl.BlockSpec((1, e), lambda g: (0, 0)),
        ],
        out_specs=pl.BlockSpec((t, e), lambda g: (g, 0)),
        out_shape=jax.ShapeDtypeStruct((n, e), jnp.float32),
    )(rows, gamma, beta)


def kernel(input_ids, token_type_ids, word_table, token_type_table,
           pos_table, gamma, beta):
    b, s = input_ids.shape
    e = word_table.shape[1]
    rows = _sc_embed(word_table, token_type_table, pos_table,
                     input_ids.reshape(-1), token_type_ids.reshape(-1))
    out = _tc_ln(rows, gamma.reshape(1, -1), beta.reshape(1, -1))
    return out.reshape(b, s, e)


# speedup vs baseline: 12.5949x; 12.5949x over previous
import functools

import jax
import jax.numpy as jnp
from jax import lax
from jax.experimental import pallas as pl
from jax.experimental.pallas import tpu as pltpu
from jax.experimental.pallas import tpu_sc as plsc

_LN_EPS = 1e-3

_NC = 2
_NS = 16
_NW = _NC * _NS


def _sc_gather_body(n_per_w, chunk, table_hbm, idx_hbm, out_hbm,
                    idx_v, buf0, buf1, gs0, gs1, ss0, ss1):
    wid = lax.axis_index("s") * _NC + lax.axis_index("c")
    base = wid * n_per_w
    pltpu.sync_copy(idx_hbm.at[pl.ds(base, n_per_w)], idx_v)

    bufs = (buf0, buf1)
    gsems = (gs0, gs1)
    ssems = (ss0, ss1)
    nchunks = n_per_w // chunk
    gathers = {}
    scatters = {}

    def start_gather(c):
        gathers[c] = pltpu.async_copy(
            table_hbm.at[idx_v.at[pl.ds(c * chunk, chunk)]],
            bufs[c % 2], gsems[c % 2])

    start_gather(0)
    for c in range(nchunks):
        if c + 1 < nchunks:
            if c - 1 >= 0:
                scatters[c - 1].wait()
            start_gather(c + 1)
        gathers[c].wait()
        scatters[c] = pltpu.async_copy(
            bufs[c % 2], out_hbm.at[pl.ds(base + c * chunk, chunk)],
            ssems[c % 2])
    for c in range(max(0, nchunks - 2), nchunks):
        scatters[c].wait()


def _sc_gather(word_table, ids_flat):
    n = ids_flat.shape[0]
    e = word_table.shape[1]
    n_per_w = n // _NW
    chunk = min(256, n_per_w)
    mesh = plsc.VectorSubcoreMesh(core_axis_name="c", subcore_axis_name="s")
    return pl.kernel(
        functools.partial(_sc_gather_body, n_per_w, chunk),
        out_type=jax.ShapeDtypeStruct((n, e), jnp.float32),
        mesh=mesh,
        scratch_types=[
            pltpu.VMEM((n_per_w,), jnp.int32),
            pltpu.VMEM((chunk, e), jnp.float32),
            pltpu.VMEM((chunk, e), jnp.float32),
            pltpu.SemaphoreType.DMA,
            pltpu.SemaphoreType.DMA,
            pltpu.SemaphoreType.DMA,
            pltpu.SemaphoreType.DMA,
        ],
    )(word_table, ids_flat)


def _tc_ln_body(rows_ref, tt_ref, tt_tab_ref, pos_ref, g_ref, b_ref, o_ref):
    ttid = tt_ref[0]
    tvals = lax.broadcasted_iota(jnp.int32, (2, ttid.shape[1]), 0)
    oh = (tvals == ttid).astype(jnp.float32)
    tte = lax.dot_general(oh, tt_tab_ref[...], (((0,), (0,)), ((), ())),
                          preferred_element_type=jnp.float32)
    x = rows_ref[...] + tte + pos_ref[...]
    mean = jnp.mean(x, axis=-1, keepdims=True)
    xc = x - mean
    var = jnp.mean(xc * xc, axis=-1, keepdims=True)
    norm = xc * lax.rsqrt(var + _LN_EPS)
    o_ref[...] = norm * g_ref[...] + b_ref[...]


def _tc_ln(rows, tt_ids, tt_table, pos_table, gamma, beta):
    n, e = rows.shape
    nb, s = tt_ids.shape
    t = 2048
    npb = s // t
    tt3 = tt_ids.reshape(nb * npb, 1, t)
    grid = (npb, nb)
    return pl.pallas_call(
        _tc_ln_body,
        grid=grid,
        in_specs=[
            pl.BlockSpec((t, e), lambda p, b: (b * npb + p, 0)),
            pl.BlockSpec((1, 1, t), lambda p, b: (b * npb + p, 0, 0)),
            pl.BlockSpec((2, e), lambda p, b: (0, 0)),
            pl.BlockSpec((t, e), lambda p, b: (p, 0)),
            pl.BlockSpec((1, e), lambda p, b: (0, 0)),
            pl.BlockSpec((1, e), lambda p, b: (0, 0)),
        ],
        out_specs=pl.BlockSpec((t, e), lambda p, b: (b * npb + p, 0)),
        out_shape=jax.ShapeDtypeStruct((n, e), jnp.float32),
    )(rows, tt3, tt_table, pos_table, gamma, beta)


def kernel(input_ids, token_type_ids, word_table, token_type_table,
           pos_table, gamma, beta):
    b, s = input_ids.shape
    e = word_table.shape[1]
    rows = _sc_gather(word_table, input_ids.reshape(-1))
    out = _tc_ln(rows, token_type_ids, token_type_table, pos_table,
                 gamma.reshape(1, -1), beta.reshape(1, -1))
    return out.reshape(b, s, e)
